# baseline (speedup 1.0000x reference)
import functools

import jax
import jax.numpy as jnp
from jax import lax
from jax.experimental import pallas as pl
from jax.experimental.pallas import tpu as pltpu
from jax.experimental.pallas import tpu_sc as plsc

N = 10000
E = 320000
D = 128
H = 128
O = 128
G = 128
L = 3

NC = 2
NS = 16
NW = NC * NS
CHUNK = 64
NBUF = 4
NCHUNKS = 160
NGROUPS = NCHUNKS // NBUF
NPAIR = NGROUPS // 2
EPAD = NW * NCHUNKS * CHUNK
ACC_ROWS = 16 * 640
ROWS_PER_TILE_OUT = 624


def _sc_agg_body(x_hbm, idx_hbm, out_hbm, ring, rows0, rows1, rows2, rows3,
                 acc_sh, gsem0, gsem1, gsem2, gsem3, ssem0, ssem1, ssem2,
                 ssem3, isem0, isem1):
    rows = (rows0, rows1, rows2, rows3)
    gsems = (gsem0, gsem1, gsem2, gsem3)
    ssems = (ssem0, ssem1, ssem2, ssem3)
    isems = (isem0, isem1)
    c = lax.axis_index("c")
    s = lax.axis_index("s")
    wid = s * NC + c

    pltpu.async_copy(idx_hbm.at[wid, 0], ring.at[0], isems[0]).wait()
    pltpu.async_copy(idx_hbm.at[wid, 1], ring.at[1], isems[1])

    for b in range(NBUF - 1):
        pltpu.async_copy(x_hbm.at[ring.at[0, 0, b]], rows[b], gsems[b])

    zero16 = jnp.zeros((16,), jnp.float32)

    def _zrow(i, _):
        def _zlane(k, _):
            rows3[i, pl.ds(k * 16, 16)] = zero16
            return 0
        return lax.fori_loop(0, D // 16, _zlane, 0)

    lax.fori_loop(0, CHUNK, _zrow, 0)

    zbase = s * 640
    for t in range(640 // CHUNK):
        pltpu.sync_copy(rows3, acc_sh.at[pl.ds(zbase + t * CHUNK, CHUNK)])

    plsc.subcore_barrier()

    pltpu.async_copy(x_hbm.at[ring.at[0, 0, NBUF - 1]], rows3,
                     gsems[NBUF - 1])

    def _phase(slot, other, nxt):
        for b in range(NBUF):
            pltpu.make_async_copy(x_hbm.at[ring.at[slot, 0, b]], rows[b],
                                  gsems[b]).wait()
            pltpu.async_copy(rows[b], acc_sh.at[ring.at[slot, 1, b]],
                             ssems[b], add=True)
        pltpu.make_async_copy(idx_hbm.at[wid, 0], ring.at[other],
                              isems[other]).wait()
        for b in range(NBUF):
            pltpu.make_async_copy(rows[b], acc_sh.at[ring.at[slot, 1, b]],
                                  ssems[b]).wait()
            pltpu.async_copy(x_hbm.at[ring.at[other, 0, b]], rows[b],
                             gsems[b])
        pltpu.async_copy(idx_hbm.at[wid, nxt], ring.at[slot], isems[slot])

    def _pair(t, _):
        _phase(0, 1, jnp.minimum(2 * t + 2, NGROUPS - 1))
        _phase(1, 0, jnp.minimum(2 * t + 3, NGROUPS - 1))
        return 0

    lax.fori_loop(0, NPAIR, _pair, 0)

    for b in range(NBUF):
        pltpu.make_async_copy(x_hbm.at[ring.at[0, 0, b]], rows[b],
                              gsems[b]).wait()
    pltpu.make_async_copy(idx_hbm.at[wid, 0], ring.at[1], isems[1]).wait()

    plsc.subcore_barrier()

    r0 = s * ROWS_PER_TILE_OUT
    pltpu.sync_copy(acc_sh.at[pl.ds(r0, ROWS_PER_TILE_OUT)],
                    out_hbm.at[c, pl.ds(r0, ROWS_PER_TILE_OUT)])

    tail = NS * ROWS_PER_TILE_OUT

    @pl.when(s == 0)
    def _emit_tail():
        pltpu.sync_copy(acc_sh.at[pl.ds(tail, N - tail)],
                        out_hbm.at[c, pl.ds(tail, N - tail)])


@functools.lru_cache(maxsize=1)
def _sc_agg_kernel():
    return functools.partial(
        pl.kernel,
        out_type=jax.ShapeDtypeStruct((NC, N, D), jnp.float32),
        mesh=plsc.VectorSubcoreMesh(core_axis_name="c", subcore_axis_name="s"),
        scratch_types=[
            pltpu.VMEM((2, 2, NBUF, CHUNK), jnp.int32),
        ] + [pltpu.VMEM((CHUNK, D), jnp.float32)] * 4 + [
            pltpu.VMEM_SHARED((ACC_ROWS, D), jnp.float32),
        ] + [pltpu.SemaphoreType.DMA] * 10,
    )(_sc_agg_body)


def _mlp_body(agg_ref, x_ref, w1_ref, b1_ref, g_ref, be_ref, w2_ref, b2_ref,
              o_ref):
    h = agg_ref[0] + agg_ref[1] + x_ref[...]
    h = lax.dot_general(h, w1_ref[...], (((1,), (0,)), ((), ())),
                        preferred_element_type=jnp.float32,
                        precision=lax.Precision.HIGHEST) + b1_ref[...]
    mu = jnp.mean(h, axis=0, keepdims=True)
    var = jnp.mean(jnp.square(h - mu), axis=0, keepdims=True)
    h = (h - mu) / jnp.sqrt(var + 1e-5) * g_ref[...] + be_ref[...]
    h = jnp.maximum(h, 0.0)
    h = lax.dot_general(h, w2_ref[...], (((1,), (0,)), ((), ())),
                        preferred_element_type=jnp.float32,
                        precision=lax.Precision.HIGHEST) + b2_ref[...]
    o_ref[...] = jnp.maximum(h, 0.0)


def _mlp_call(agg, x, w1, b1, g, be, w2, b2):
    return pl.pallas_call(
        _mlp_body,
        out_shape=jax.ShapeDtypeStruct((N, H), jnp.float32),
    )(agg, x, w1, b1, g, be, w2, b2)


def _pool_body(x_ref, batch_ref, w1_ref, b1_ref, w2_ref, b2_ref, o_ref):
    gi = lax.broadcasted_iota(jnp.int32, (G, N), 0)
    oh = jnp.where(batch_ref[...] == gi, 1.0, 0.0).astype(jnp.float32)
    pooled = lax.dot_general(oh, x_ref[...], (((1,), (0,)), ((), ())),
                             preferred_element_type=jnp.float32,
                             precision=lax.Precision.HIGHEST)
    h = lax.dot_general(pooled, w1_ref[...], (((1,), (0,)), ((), ())),
                        preferred_element_type=jnp.float32,
                        precision=lax.Precision.HIGHEST) + b1_ref[...]
    h = jnp.maximum(h, 0.0)
    o_ref[...] = lax.dot_general(h, w2_ref[...], (((1,), (0,)), ((), ())),
                                 preferred_element_type=jnp.float32,
                                 precision=lax.Precision.HIGHEST) + b2_ref[...]


def _pool_call(x, batch_row, w1, b1, w2, b2):
    return pl.pallas_call(
        _pool_body,
        out_shape=jax.ShapeDtypeStruct((G, O), jnp.float32),
    )(x, batch_row, w1, b1, w2, b2)


def kernel(x, edge_index, batch, conv_W1, conv_b1, conv_gamma, conv_beta,
           conv_W2, conv_b2, mlp_W1, mlp_b1, mlp_W2, mlp_b2):
    src = edge_index[0]
    dst = edge_index[1]
    pad = EPAD - E
    ar = jnp.arange(pad, dtype=jnp.int32)
    src_p = jnp.concatenate([src, ar % N])
    dst_p = jnp.concatenate([dst, N + (ar % (ACC_ROWS - N))])
    src4 = src_p.reshape(NW, NGROUPS, NBUF, CHUNK)
    dst4 = dst_p.reshape(NW, NGROUPS, NBUF, CHUNK)
    idxcomb = jnp.stack([src4, dst4], axis=2)
    batch_row = batch.reshape(1, N)

    for l in range(L):
        parts = _sc_agg_kernel()(x, idxcomb)
        x = _mlp_call(parts, x, conv_W1[l], conv_b1[l].reshape(1, H),
                      conv_gamma[l].reshape(1, H), conv_beta[l].reshape(1, H),
                      conv_W2[l], conv_b2[l].reshape(1, H))

    return _pool_call(x, batch_row, mlp_W1, mlp_b1.reshape(1, H),
                      mlp_W2, mlp_b2.reshape(1, O))


# speedup vs baseline: 1.1017x; 1.1017x over previous
import functools

import jax
import jax.numpy as jnp
from jax import lax
from jax.experimental import pallas as pl
from jax.experimental.pallas import tpu as pltpu
from jax.experimental.pallas import tpu_sc as plsc

N = 10000
E = 320000
D = 128
H = 128
O = 128
G = 128
L = 3

NC = 2
NS = 16
NW = NC * NS
CHUNK = 64
NBUF = 4
NCHUNKS = 160
NGROUPS = NCHUNKS // NBUF
NPAIR = NGROUPS // 2
EPAD = NW * NCHUNKS * CHUNK
ACC_ROWS = 16 * 640
ROWS_PER_TILE_OUT = 624


def _sc_agg_body(x_hbm, idx_hbm, out_hbm, ring, rows0, rows1, rows2, rows3,
                 acc_sh, gsem0, gsem1, gsem2, gsem3, ssem0, ssem1, ssem2,
                 ssem3, isem0, isem1):
    rows = (rows0, rows1, rows2, rows3)
    gsems = (gsem0, gsem1, gsem2, gsem3)
    ssems = (ssem0, ssem1, ssem2, ssem3)
    isems = (isem0, isem1)
    c = lax.axis_index("c")
    s = lax.axis_index("s")
    wid = s * NC + c

    pltpu.async_copy(idx_hbm.at[wid, 0], ring.at[0], isems[0]).wait()
    pltpu.async_copy(idx_hbm.at[wid, 1], ring.at[1], isems[1])

    for b in range(NBUF - 1):
        pltpu.async_copy(x_hbm.at[ring.at[0, 0, b]], rows[b], gsems[b])

    zero16 = jnp.zeros((16,), jnp.float32)

    def _zrow(i, _):
        def _zlane(k, _):
            rows3[i, pl.ds(k * 16, 16)] = zero16
            return 0
        return lax.fori_loop(0, D // 16, _zlane, 0)

    lax.fori_loop(0, CHUNK, _zrow, 0)

    zbase = s * 640
    for t in range(640 // CHUNK):
        pltpu.sync_copy(rows3, acc_sh.at[pl.ds(zbase + t * CHUNK, CHUNK)])

    plsc.subcore_barrier()

    pltpu.async_copy(x_hbm.at[ring.at[0, 0, NBUF - 1]], rows3,
                     gsems[NBUF - 1])

    def _phase(slot, other, nxt):
        for b in range(NBUF):
            pltpu.make_async_copy(x_hbm.at[ring.at[slot, 0, b]], rows[b],
                                  gsems[b]).wait()
            pltpu.sync_copy(rows[b], acc_sh.at[ring.at[slot, 1, b]],
                            add=True)
            if b == 0:
                pltpu.make_async_copy(idx_hbm.at[wid, 0], ring.at[other],
                                      isems[other]).wait()
            pltpu.async_copy(x_hbm.at[ring.at[other, 0, b]], rows[b],
                             gsems[b])
        pltpu.async_copy(idx_hbm.at[wid, nxt], ring.at[slot], isems[slot])

    def _pair(t, _):
        _phase(0, 1, jnp.minimum(2 * t + 2, NGROUPS - 1))
        _phase(1, 0, jnp.minimum(2 * t + 3, NGROUPS - 1))
        return 0

    lax.fori_loop(0, NPAIR, _pair, 0)

    for b in range(NBUF):
        pltpu.make_async_copy(x_hbm.at[ring.at[0, 0, b]], rows[b],
                              gsems[b]).wait()
    pltpu.make_async_copy(idx_hbm.at[wid, 0], ring.at[1], isems[1]).wait()

    plsc.subcore_barrier()

    r0 = s * ROWS_PER_TILE_OUT
    pltpu.sync_copy(acc_sh.at[pl.ds(r0, ROWS_PER_TILE_OUT)],
                    out_hbm.at[c, pl.ds(r0, ROWS_PER_TILE_OUT)])

    tail = NS * ROWS_PER_TILE_OUT

    @pl.when(s == 0)
    def _emit_tail():
        pltpu.sync_copy(acc_sh.at[pl.ds(tail, N - tail)],
                        out_hbm.at[c, pl.ds(tail, N - tail)])


@functools.lru_cache(maxsize=1)
def _sc_agg_kernel():
    return functools.partial(
        pl.kernel,
        out_type=jax.ShapeDtypeStruct((NC, N, D), jnp.float32),
        mesh=plsc.VectorSubcoreMesh(core_axis_name="c", subcore_axis_name="s"),
        scratch_types=[
            pltpu.VMEM((2, 2, NBUF, CHUNK), jnp.int32),
        ] + [pltpu.VMEM((CHUNK, D), jnp.float32)] * 4 + [
            pltpu.VMEM_SHARED((ACC_ROWS, D), jnp.float32),
        ] + [pltpu.SemaphoreType.DMA] * 10,
    )(_sc_agg_body)


def _mlp_body(agg_ref, x_ref, w1_ref, b1_ref, g_ref, be_ref, w2_ref, b2_ref,
              o_ref):
    h = agg_ref[0] + agg_ref[1] + x_ref[...]
    h = lax.dot_general(h, w1_ref[...], (((1,), (0,)), ((), ())),
                        preferred_element_type=jnp.float32,
                        precision=lax.Precision.HIGHEST) + b1_ref[...]
    mu = jnp.mean(h, axis=0, keepdims=True)
    var = jnp.mean(jnp.square(h - mu), axis=0, keepdims=True)
    h = (h - mu) / jnp.sqrt(var + 1e-5) * g_ref[...] + be_ref[...]
    h = jnp.maximum(h, 0.0)
    h = lax.dot_general(h, w2_ref[...], (((1,), (0,)), ((), ())),
                        preferred_element_type=jnp.float32,
                        precision=lax.Precision.HIGHEST) + b2_ref[...]
    o_ref[...] = jnp.maximum(h, 0.0)


def _mlp_call(agg, x, w1, b1, g, be, w2, b2):
    return pl.pallas_call(
        _mlp_body,
        out_shape=jax.ShapeDtypeStruct((N, H), jnp.float32),
    )(agg, x, w1, b1, g, be, w2, b2)


def _pool_body(x_ref, batch_ref, w1_ref, b1_ref, w2_ref, b2_ref, o_ref):
    gi = lax.broadcasted_iota(jnp.int32, (G, N), 0)
    oh = jnp.where(batch_ref[...] == gi, 1.0, 0.0).astype(jnp.float32)
    pooled = lax.dot_general(oh, x_ref[...], (((1,), (0,)), ((), ())),
                             preferred_element_type=jnp.float32,
                             precision=lax.Precision.HIGHEST)
    h = lax.dot_general(pooled, w1_ref[...], (((1,), (0,)), ((), ())),
                        preferred_element_type=jnp.float32,
                        precision=lax.Precision.HIGHEST) + b1_ref[...]
    h = jnp.maximum(h, 0.0)
    o_ref[...] = lax.dot_general(h, w2_ref[...], (((1,), (0,)), ((), ())),
                                 preferred_element_type=jnp.float32,
                                 precision=lax.Precision.HIGHEST) + b2_ref[...]


def _pool_call(x, batch_row, w1, b1, w2, b2):
    return pl.pallas_call(
        _pool_body,
        out_shape=jax.ShapeDtypeStruct((G, O), jnp.float32),
    )(x, batch_row, w1, b1, w2, b2)


def kernel(x, edge_index, batch, conv_W1, conv_b1, conv_gamma, conv_beta,
           conv_W2, conv_b2, mlp_W1, mlp_b1, mlp_W2, mlp_b2):
    src = edge_index[0]
    dst = edge_index[1]
    pad = EPAD - E
    ar = jnp.arange(pad, dtype=jnp.int32)
    src_p = jnp.concatenate([src, ar % N])
    dst_p = jnp.concatenate([dst, N + (ar % (ACC_ROWS - N))])
    src4 = src_p.reshape(NW, NGROUPS, NBUF, CHUNK)
    dst4 = dst_p.reshape(NW, NGROUPS, NBUF, CHUNK)
    idxcomb = jnp.stack([src4, dst4], axis=2)
    batch_row = batch.reshape(1, N)

    for l in range(L):
        parts = _sc_agg_kernel()(x, idxcomb)
        x = _mlp_call(parts, x, conv_W1[l], conv_b1[l].reshape(1, H),
                      conv_gamma[l].reshape(1, H), conv_beta[l].reshape(1, H),
                      conv_W2[l], conv_b2[l].reshape(1, H))

    return _pool_call(x, batch_row, mlp_W1, mlp_b1.reshape(1, H),
                      mlp_W2, mlp_b2.reshape(1, O))
